# baseline (speedup 1.0000x reference)
import jax
import jax.numpy as jnp
from jax.experimental import pallas as pl

CLS = 16
N_AGENT = 96
N_KNN = 20
RADIUS = 0.2


def _c2(x, W, b):
    return jnp.einsum('oc,bcmk->bomk', W, x) + b[None, :, None, None]


def _c1(x, W, b):
    return jnp.einsum('oc,bcm->bom', W, x) + b[None, :, None]


def _group(feat, idx):
    B, C, N = feat.shape
    M, K = idx.shape[1], idx.shape[2]
    idxb = jnp.broadcast_to(idx.reshape(B, 1, M * K), (B, C, M * K))
    return jnp.take_along_axis(feat, idxb, axis=2).reshape(B, C, M, K)


def _combine_kernel(agent_ref, trans_ref, out_ref):
    out_ref[...] = agent_ref[...] + trans_ref[...]


def kernel(pcd_coarse, trans_cord, k_prev, cW1, cB1, cW2, cB2, lW1, lB1, lW2, lB2, dW1, dB1, dW2, dB2, dW3, dB3, eW1, eB1, eW2, eB2, eW3, eB3):
    B, C, N = pcd_coarse.shape
    trans_dist = jnp.sum(jnp.transpose(trans_cord, (0, 2, 1)) ** 2, axis=2)
    _, idx_agent = jax.lax.top_k(trans_dist, N_AGENT)
    pcd_agent = jnp.take_along_axis(pcd_coarse, idx_agent[:, None, :], axis=2)
    ref = pcd_coarse[:, 0:3, :]
    q = pcd_agent[:, 0:3, :]
    qT = jnp.transpose(q, (0, 2, 1))
    rT = jnp.transpose(ref, (0, 2, 1))
    d2 = jnp.sum(qT ** 2, -1)[:, :, None] + jnp.sum(rT ** 2, -1)[:, None, :] - 2.0 * jnp.einsum('bqd,bnd->bqn', qT, rT)
    _, idx_knn = jax.lax.top_k(-d2, N_KNN)
    cord_patch = _group(ref, idx_knn)
    label_patch = _group(pcd_coarse[:, 3:, :], idx_knn)
    cord_patch = cord_patch - cord_patch[:, :, :, 0:1]
    label_patch = label_patch - label_patch[:, :, :, 0:1]
    cf = _c2(jax.nn.relu(_c2(cord_patch, cW1, cB1)), cW2, cB2)
    lf = _c2(jax.nn.relu(_c2(label_patch, lW1, lB1)), lW2, lB2)
    cf = jnp.max(cf, axis=3)
    lf = jnp.max(lf, axis=3)
    h = jax.nn.relu(cf)
    h = jax.nn.relu(_c1(h, dW1, dB1))
    h = jax.nn.relu(_c1(h, dW2, dB2))
    child_cmp = jnp.tanh(_c1(h, dW3, dB3)) * RADIUS
    g = jax.nn.relu(lf)
    g = jax.nn.relu(_c1(g, eW1, eB1))
    g = jax.nn.relu(_c1(g, eW2, eB2))
    child_label = jnp.tanh(_c1(g, eW3, eB3)) * RADIUS
    local_trans = jnp.concatenate([child_cmp, child_label], axis=1)
    pcd_local_agents = pl.pallas_call(
        _combine_kernel,
        out_shape=jax.ShapeDtypeStruct(pcd_agent.shape, pcd_agent.dtype),
    )(pcd_agent, local_trans)
    pcd_local = jnp.concatenate([pcd_coarse, pcd_local_agents], axis=2)
    k_prev_agent = jnp.take_along_axis(k_prev, idx_agent[:, None, :], axis=2)
    k_prev_out = jnp.concatenate([k_prev, k_prev_agent], axis=2)
    return (pcd_local, k_prev_out)


# speedup vs baseline: 25.6876x; 25.6876x over previous
import functools

import jax
import jax.numpy as jnp
from jax import lax
from jax.experimental import pallas as pl
from jax.experimental.pallas import tpu as pltpu
from jax.experimental.pallas import tpu_sc as plsc

CLS = 16
N_AGENT = 96
N_KNN = 20
RADIUS = 0.2

B = 16
CH = 3 + CLS
KPC = 64
N = 16384
NQ = B * N_AGENT
KPAD = 32
HALF_Q = N_AGENT // 2
NROW = 64
NCOLV = 16
BIG = 1e30

_i32 = jnp.int32
_f32 = jnp.float32


def _iota16():
    return lax.iota(_i32, 16)


def _splat_f(x):
    return jnp.full((16,), x, _f32)


def _splat_i(x):
    return jnp.full((16,), x, _i32)


def _pass1(xr, yr, zr, dref, mref, qx, qy, qz, scale):
    iota = _iota16()
    init = tuple(_splat_f(BIG) for _ in range(16))

    @pl.loop(0, NROW, init_carry=init)
    def mins(r, M):
        rb = pl.multiple_of(r * 256, 256)
        out = []
        for v in range(16):
            off = rb + v * 16
            x = xr[pl.ds(off, 16)]
            y = yr[pl.ds(off, 16)]
            z = zr[pl.ds(off, 16)]
            dx = x - qx
            dy = y - qy
            dz = z - qz
            d = (dx * dx + dy * dy + dz * dz) * scale
            dref[pl.ds(off, 16)] = d
            out.append(jnp.minimum(M[v], d))
        return tuple(out)

    SM = _splat_f(BIG)
    for v in range(16):
        mref[pl.ds(v * 16, 16)] = mins[v]
        SM = jnp.where(iota == v, _splat_f(jnp.min(mins[v])), SM)
    return SM


def _extract(dref, mref, idxref, base, SM0, nrounds):
    iota = _iota16()

    @pl.loop(0, nrounds, init_carry=SM0)
    def SMf(j, SM):
        gm = jnp.min(SM)
        gmv = _splat_f(gm)
        vstar = jnp.min(jnp.where(SM == gmv, iota, _splat_i(999)))
        moff = pl.multiple_of(vstar * 16, 16)
        mv = mref[pl.ds(moff, 16)]
        lane = jnp.min(jnp.where(mv == gmv, iota, _splat_i(999)))
        col = vstar * 16 + lane
        gks = []
        cands = []
        for k in range(4):
            ridx = iota + (16 * k)
            gk = plsc.load_gather(dref, [col + ridx * 256])
            gks.append((ridx, gk))
            cands.append(jnp.where(gk == gmv, ridx, _splat_i(999)))
        rstar = jnp.min(jnp.minimum(jnp.minimum(cands[0], cands[1]),
                                    jnp.minimum(cands[2], cands[3])))
        n = rstar * 256 + col
        m0 = iota == 0
        plsc.store_scatter(idxref, [_splat_i(base + j)], _splat_i(n), mask=m0)
        plsc.store_scatter(dref, [_splat_i(n)], _splat_f(BIG), mask=m0)
        news = []
        for k in range(4):
            ridx, gk = gks[k]
            news.append(jnp.where(ridx == rstar, _splat_f(BIG), gk))
        ncm = jnp.min(jnp.minimum(jnp.minimum(news[0], news[1]),
                                  jnp.minimum(news[2], news[3])))
        mv2 = jnp.where(iota == lane, _splat_f(ncm), mv)
        mref[pl.ds(moff, 16)] = mv2
        return jnp.where(iota == vstar, _splat_f(jnp.min(mv2)), SM)


def _sc_body(pcd_flat, trans_flat, kprev_flat,
             patches, pcd_agentT, kprevT,
             xbuf, ybuf, zbuf, dbuf, mbuf, agidx, knnb,
             cordp, labelp, gidx, gdst, sem):
    iota = _iota16()
    b = lax.axis_index("s")
    h = lax.axis_index("c")
    zero = _splat_f(0.0)

    for ci, buf in enumerate((xbuf, ybuf, zbuf)):
        pltpu.sync_copy(trans_flat.at[pl.ds((b * 3 + ci) * N, N)], buf)
    SMt = _pass1(xbuf, ybuf, zbuf, dbuf, mbuf, zero, zero, zero,
                 _splat_f(-1.0))
    _extract(dbuf, mbuf, agidx, 0, SMt, N_AGENT)

    for ci, buf in enumerate((xbuf, ybuf, zbuf)):
        pltpu.sync_copy(pcd_flat.at[pl.ds((b * CH + ci) * N, N)], buf)

    @pl.loop(0, 32)
    def _kp(ci):
        base = (b * KPC + h * 32 + ci) * N
        for vv in range(6):
            iv = agidx[pl.ds(vv * 16, 16)]
            gidx[ci, pl.ds(vv * 16, 16)] = iv + base
        pltpu.async_copy(kprev_flat.at[gidx.at[ci]],
                         gdst.at[pl.ds(ci * N_AGENT, N_AGENT)], sem).wait()

    pltpu.sync_copy(gdst.at[pl.ds(0, 32 * N_AGENT)],
                    kprevT.at[pl.ds(b * KPC * N_AGENT + h * 32 * N_AGENT,
                                    32 * N_AGENT)])

    @pl.loop(0, 10)
    def _pa(ci):
        base = (b * CH + h * 9 + ci) * N
        for vv in range(6):
            iv = agidx[pl.ds(vv * 16, 16)]
            gidx[ci, pl.ds(vv * 16, 16)] = iv + base
        pltpu.async_copy(pcd_flat.at[gidx.at[ci]],
                         gdst.at[pl.ds(ci * N_AGENT, N_AGENT)], sem).wait()

    @pl.loop(0, 10)
    def _paout(ci):
        pltpu.sync_copy(
            gdst.at[pl.ds(ci * N_AGENT, N_AGENT)],
            pcd_agentT.at[pl.ds((h * 9 + ci) * NQ + b * N_AGENT, N_AGENT)])

    @pl.loop(0, HALF_Q)
    def _knn(q):
        aq = h * HALF_Q + q
        grp = pl.multiple_of((aq // 16) * 16, 16)
        av = agidx[pl.ds(grp, 16)]
        qn = jnp.sum(jnp.where(iota == (aq % 16), av, _splat_i(0)))
        qi = _splat_i(qn)
        qxv = plsc.load_gather(xbuf, [qi])
        qyv = plsc.load_gather(ybuf, [qi])
        qzv = plsc.load_gather(zbuf, [qi])
        SM = _pass1(xbuf, ybuf, zbuf, dbuf, mbuf, qxv, qyv, qzv,
                    _splat_f(1.0))
        knnb[pl.ds(q * KPAD + 16, 16)] = _splat_i(0)
        _extract(dbuf, mbuf, knnb, q * KPAD, SM, N_KNN)
        a0 = knnb[pl.ds(q * KPAD, 16)]
        a1 = knnb[pl.ds(q * KPAD + 16, 16)]
        for ci, buf in enumerate((xbuf, ybuf, zbuf)):
            cordp[pl.ds(ci * (HALF_Q * KPAD) + q * KPAD, 16)] = \
                plsc.load_gather(buf, [a0])
            cordp[pl.ds(ci * (HALF_Q * KPAD) + q * KPAD + 16, 16)] = \
                plsc.load_gather(buf, [a1])

    colbase = (b * N_AGENT + h * HALF_Q) * KPAD
    for ci in range(3):
        pltpu.sync_copy(cordp.at[pl.ds(ci * (HALF_Q * KPAD), HALF_Q * KPAD)],
                        patches.at[pl.ds(ci * (NQ * KPAD) + colbase,
                                         HALF_Q * KPAD)])

    @pl.loop(0, CLS)
    def _lab(ci):
        pltpu.sync_copy(pcd_flat.at[pl.ds((b * CH + 3 + ci) * N, N)], xbuf)

        @pl.loop(0, HALF_Q)
        def _labq(q):
            a0 = knnb[pl.ds(q * KPAD, 16)]
            a1 = knnb[pl.ds(q * KPAD + 16, 16)]
            labelp[pl.ds(ci * (HALF_Q * KPAD) + q * KPAD, 16)] = \
                plsc.load_gather(xbuf, [a0])
            labelp[pl.ds(ci * (HALF_Q * KPAD) + q * KPAD + 16, 16)] = \
                plsc.load_gather(xbuf, [a1])

    @pl.loop(0, CLS)
    def _labout(ci):
        pltpu.sync_copy(labelp.at[pl.ds(ci * (HALF_Q * KPAD), HALF_Q * KPAD)],
                        patches.at[pl.ds((3 + ci) * (NQ * KPAD) + colbase,
                                         HALF_Q * KPAD)])


_sc_knn = functools.partial(
    pl.kernel,
    out_type=(
        jax.ShapeDtypeStruct((CH * NQ * KPAD,), _f32),
        jax.ShapeDtypeStruct((CH * NQ,), _f32),
        jax.ShapeDtypeStruct((B * KPC * N_AGENT,), _f32),
    ),
    mesh=plsc.VectorSubcoreMesh(core_axis_name="c", subcore_axis_name="s"),
    compiler_params=pltpu.CompilerParams(needs_layout_passes=False),
    scratch_types=[
        pltpu.VMEM((N,), _f32),
        pltpu.VMEM((N,), _f32),
        pltpu.VMEM((N,), _f32),
        pltpu.VMEM((N,), _f32),
        pltpu.VMEM((256,), _f32),
        pltpu.VMEM((N_AGENT,), _i32),
        pltpu.VMEM((HALF_Q * KPAD,), _i32),
        pltpu.VMEM((3 * HALF_Q * KPAD,), _f32),
        pltpu.VMEM((CLS * HALF_Q * KPAD,), _f32),
        pltpu.VMEM((32, N_AGENT), _i32),
        pltpu.VMEM((32 * N_AGENT,), _f32),
        pltpu.SemaphoreType.DMA,
    ],
)(_sc_body)


QB = 128


def _tc_body(patches_ref, pcd_agentT_ref,
             cW1r, cB1r, cW2r, cB2r, lW1r, lB1r, lW2r, lB2r,
             dW1r, dB1r, dW2r, dB2r, dW3r, dB3r,
             eW1r, eB1r, eW2r, eB2r, eW3r, eB3r, out_ref):
    X = patches_ref[...].reshape(CH, QB, KPAD)
    X = X - X[:, :, 0:1]
    cord = X[0:3].reshape(3, QB * KPAD)
    label = X[3:CH].reshape(CLS, QB * KPAD)

    def mm(Wr, x):
        return jnp.dot(Wr[...], x, preferred_element_type=jnp.float32)

    kmask = lax.broadcasted_iota(_i32, (1, 1, KPAD), 2) < N_KNN

    def patch_stack(x, W1r, b1r, W2r, b2r):
        hh = jnp.maximum(mm(W1r, x) + b1r[...][:, None], 0.0)
        ff = mm(W2r, hh) + b2r[...][:, None]
        ff = ff.reshape(64, QB, KPAD)
        ff = jnp.where(kmask, ff, -BIG)
        return jnp.max(ff, axis=2)

    cf = patch_stack(cord, cW1r, cB1r, cW2r, cB2r)
    lf = patch_stack(label, lW1r, lB1r, lW2r, lB2r)

    hd = jnp.maximum(cf, 0.0)
    hd = jnp.maximum(mm(dW1r, hd) + dB1r[...][:, None], 0.0)
    hd = jnp.maximum(mm(dW2r, hd) + dB2r[...][:, None], 0.0)
    child_cmp = jnp.tanh(mm(dW3r, hd) + dB3r[...][:, None]) * RADIUS

    he = jnp.maximum(lf, 0.0)
    he = jnp.maximum(mm(eW1r, he) + eB1r[...][:, None], 0.0)
    he = jnp.maximum(mm(eW2r, he) + eB2r[...][:, None], 0.0)
    child_label = jnp.tanh(mm(eW3r, he) + eB3r[...][:, None]) * RADIUS

    local_trans = jnp.concatenate([child_cmp, child_label], axis=0)
    out_ref[...] = pcd_agentT_ref[...] + local_trans


def kernel(pcd_coarse, trans_cord, k_prev, cW1, cB1, cW2, cB2, lW1, lB1,
           lW2, lB2, dW1, dB1, dW2, dB2, dW3, dB3, eW1, eB1, eW2, eB2,
           eW3, eB3):
    patches, pcd_agentT, kprev_agent = _sc_knn(
        pcd_coarse.reshape(-1), trans_cord.reshape(-1), k_prev.reshape(-1))

    wspecs = [pl.BlockSpec(w.shape, lambda i, nd=w.ndim: (0,) * nd)
              for w in (cW1, cB1, cW2, cB2, lW1, lB1, lW2, lB2,
                        dW1, dB1, dW2, dB2, dW3, dB3,
                        eW1, eB1, eW2, eB2, eW3, eB3)]
    agentsT = pl.pallas_call(
        _tc_body,
        grid=(NQ // QB,),
        in_specs=[pl.BlockSpec((CH, QB * KPAD), lambda i: (0, i)),
                  pl.BlockSpec((CH, QB), lambda i: (0, i))] + wspecs,
        out_specs=pl.BlockSpec((CH, QB), lambda i: (0, i)),
        out_shape=jax.ShapeDtypeStruct((CH, NQ), _f32),
    )(patches.reshape(CH, NQ * KPAD), pcd_agentT.reshape(CH, NQ),
      cW1, cB1, cW2, cB2, lW1, lB1, lW2, lB2,
      dW1, dB1, dW2, dB2, dW3, dB3, eW1, eB1, eW2, eB2, eW3, eB3)

    agents = agentsT.reshape(CH, B, N_AGENT).transpose(1, 0, 2)
    pcd_local = jnp.concatenate([pcd_coarse, agents], axis=2)
    kpa = kprev_agent.reshape(B, KPC, N_AGENT)
    k_prev_out = jnp.concatenate([k_prev, kpa], axis=2)
    return (pcd_local, k_prev_out)
